# baseline (device time: 43596 ns/iter reference)
import numpy as np

import jax
import jax.numpy as jnp
from jax import lax
from jax.experimental import pallas as pl
from jax.experimental.pallas import tpu as pltpu

P = 16
M = 4096
M_BLK = M // P
K = 4096
KC = 256
N = 8192
SPS = 2
NSTEP = P // SPS

_PLANE = [(0, 0), (1, 0), (1, 1), (0, 1)]


def _coords(p):
    x, y = _PLANE[p % 4]
    return (x, y, p // 4)


def _dist(a, b):
    ax, ay, az = _coords(a)
    bx, by, bz = _coords(b)
    return abs(ax - bx) + abs(ay - by) + abs(az - bz)


_ORDER = np.array(
    [sorted(range(P), key=lambda s, i=i: (_dist(i, s), (s - i) % P))
     for i in range(P)],
    np.int32,
)


def kernel(x, w_mat, scale_x, scale_w):
    my_pos = lax.axis_index("i")
    order_row = jnp.asarray(_ORDER)[my_pos]

    def body(x_ref, w_hbm, sx_ref, sw_ref, ord_ref, out_ref,
             xg_ref, wbuf, send_sems, recv_sems, w_sems):
        my = lax.axis_index("i")

        xg_ref[:, pl.ds(my * M_BLK, M_BLK)] = x_ref[pl.ds(my * M_BLK, M_BLK), :]

        def w_copy(t, slot, c):
            s = ord_ref[t * SPS + c]
            return pltpu.make_async_copy(
                w_hbm.at[pl.ds(s * KC, KC), :],
                wbuf.at[slot, pl.ds(c * KC, KC), :],
                w_sems.at[slot, c],
            )

        def start_w(t, slot):
            for c in range(SPS):
                w_copy(t, slot, c).start()

        def wait_w(t, slot):
            for c in range(SPS):
                w_copy(t, slot, c).wait()

        start_w(0, 0)

        sends = []
        for d in range(1, P):
            dst = lax.rem(my + d, P)
            r = pltpu.make_async_remote_copy(
                src_ref=x_ref.at[pl.ds(dst * M_BLK, M_BLK), :],
                dst_ref=xg_ref.at[:, pl.ds(my * M_BLK, M_BLK)],
                send_sem=send_sems.at[d],
                recv_sem=recv_sems.at[d],
                device_id=(dst,),
                device_id_type=pl.DeviceIdType.MESH,
            )
            r.start()
            sends.append(r)

        for t in range(NSTEP):
            slot = t % 2
            if t + 1 < NSTEP:
                start_w(t + 1, (t + 1) % 2)
            for c in range(SPS):
                s = ord_ref[t * SPS + c]
                d = lax.rem(my - s + P, P)

                @pl.when(d != 0)
                def _wait_chunk(s=s, d=d):
                    pltpu.make_async_remote_copy(
                        src_ref=x_ref.at[pl.ds(0, M_BLK), :],
                        dst_ref=xg_ref.at[:, pl.ds(s * M_BLK, M_BLK)],
                        send_sem=send_sems.at[0],
                        recv_sem=recv_sems.at[d],
                        device_id=(my,),
                        device_id_type=pl.DeviceIdType.MESH,
                    ).wait_recv()
            wait_w(t, slot)

            xc = jnp.concatenate(
                [xg_ref[:, pl.ds(ord_ref[t * SPS + c] * M_BLK, M_BLK)]
                 for c in range(SPS)],
                axis=1,
            ).astype(jnp.bfloat16)
            wb = wbuf[slot].astype(jnp.bfloat16)
            part = lax.dot_general(
                xc, wb,
                dimension_numbers=(((1,), (0,)), ((), ())),
                preferred_element_type=jnp.float32,
            )
            if t == 0:
                out_ref[:, :] = part
            else:
                out_ref[:, :] = out_ref[:, :] + part

        for r in sends:
            r.wait_send()

        y = out_ref[:, :] * (sx_ref[0] * sw_ref[0])
        out_ref[:, :] = y * 0.5 * (1.0 + jnp.tanh(0.5 * y))

    return pl.pallas_call(
        body,
        in_specs=[
            pl.BlockSpec(memory_space=pltpu.VMEM),
            pl.BlockSpec(memory_space=pl.ANY),
            pl.BlockSpec(memory_space=pltpu.SMEM),
            pl.BlockSpec(memory_space=pltpu.SMEM),
            pl.BlockSpec(memory_space=pltpu.SMEM),
        ],
        out_specs=pl.BlockSpec(memory_space=pltpu.VMEM),
        out_shape=jax.ShapeDtypeStruct((M_BLK, N), jnp.float32),
        scratch_shapes=[
            pltpu.VMEM((M_BLK, K), jnp.int8),
            pltpu.VMEM((2, SPS * KC, N), jnp.int8),
            pltpu.SemaphoreType.DMA((P,)),
            pltpu.SemaphoreType.DMA((P,)),
            pltpu.SemaphoreType.DMA((2, SPS)),
        ],
    )(x, w_mat, scale_x, scale_w, order_row)
